# baseline (speedup 1.0000x reference)
import functools

import jax
import jax.numpy as jnp
from jax import lax
from jax.experimental import pallas as pl
from jax.experimental.pallas import tpu as pltpu
from jax.experimental.pallas import tpu_sc as plsc

N_CODES = 8192
DIM = 256
BETA = 0.25

R_BLK = 1024
C_BLK = 1024
N_CB = N_CODES // C_BLK


def _dist_argmin_body(zp_ref, w_ref, idx_ref, mind_ref, best_ref, bidx_ref):
    cb = pl.program_id(1)
    zi = zp_ref[...]
    wb = w_ref[...]
    s = lax.dot_general(zi, wb, (((1,), (1,)), ((), ())),
                        preferred_element_type=jnp.float32)
    zsq = jnp.sum(zi * zi, axis=1, keepdims=True)
    wsq = jnp.sum(wb * wb, axis=1, keepdims=True).reshape(1, C_BLK)
    d = (zsq + wsq) - 2.0 * s

    bmin = jnp.min(d, axis=1, keepdims=True)
    col = lax.broadcasted_iota(jnp.int32, (R_BLK, C_BLK), 1)
    barg = jnp.min(jnp.where(d == bmin, col, N_CODES), axis=1,
                   keepdims=True) + cb * C_BLK

    @pl.when(cb == 0)
    def _init():
        best_ref[...] = bmin
        bidx_ref[...] = barg

    @pl.when(cb > 0)
    def _update():
        prev = best_ref[...]
        take = bmin < prev
        best_ref[...] = jnp.where(take, bmin, prev)
        bidx_ref[...] = jnp.where(take, barg, bidx_ref[...])

    @pl.when(cb == N_CB - 1)
    def _emit():
        idx_ref[0] = bidx_ref[...]
        mind_ref[0] = best_ref[...]


def _dist_argmin(zp_flat, w):
    n_rb = zp_flat.shape[0] // R_BLK
    return pl.pallas_call(
        _dist_argmin_body,
        grid=(n_rb, N_CB),
        in_specs=[
            pl.BlockSpec((R_BLK, DIM), lambda rb, cb: (rb, 0)),
            pl.BlockSpec((C_BLK, DIM), lambda rb, cb: (cb, 0)),
        ],
        out_specs=[
            pl.BlockSpec((1, R_BLK, 1), lambda rb, cb: (rb, 0, 0)),
            pl.BlockSpec((1, R_BLK, 1), lambda rb, cb: (rb, 0, 0)),
        ],
        out_shape=[
            jax.ShapeDtypeStruct((n_rb, R_BLK, 1), jnp.int32),
            jax.ShapeDtypeStruct((n_rb, R_BLK, 1), jnp.float32),
        ],
        scratch_shapes=[
            pltpu.VMEM((R_BLK, 1), jnp.float32),
            pltpu.VMEM((R_BLK, 1), jnp.int32),
        ],
        compiler_params=pltpu.CompilerParams(
            dimension_semantics=("parallel", "arbitrary")),
    )(zp_flat, w)


def _sc_gather(w, idx):
    info = plsc.get_sparse_core_info()
    nw = info.num_cores * info.num_subcores
    n = idx.shape[0]
    b_per_w = n // nw
    mesh = plsc.VectorSubcoreMesh(core_axis_name="c", subcore_axis_name="s")

    @functools.partial(
        pl.kernel, mesh=mesh,
        out_type=jax.ShapeDtypeStruct((n, DIM), jnp.float32),
        scratch_types=[
            pltpu.VMEM((b_per_w,), jnp.int32),
            pltpu.VMEM((b_per_w, DIM), jnp.float32),
            pltpu.SemaphoreType.DMA,
        ],
    )
    def gather_k(table_hbm, idx_hbm, out_hbm, idx_v, rows_v, sem):
        wid = lax.axis_index("s") * info.num_cores + lax.axis_index("c")
        base = wid * b_per_w
        pltpu.sync_copy(idx_hbm.at[pl.ds(base, b_per_w)], idx_v)
        pltpu.async_copy(table_hbm.at[idx_v], rows_v, sem).wait()
        pltpu.sync_copy(rows_v, out_hbm.at[pl.ds(base, b_per_w)])

    return gather_k(w, idx)


def kernel(z, W):
    b, c, h, w_ = z.shape
    n = b * h * w_
    zp = jnp.transpose(z, (0, 2, 3, 1))
    zp_flat = zp.reshape(n, DIM)

    idx3, mind3 = _dist_argmin(zp_flat, W)
    idx = idx3.reshape(n)

    zq_flat = _sc_gather(W, idx)

    m = jnp.sum(mind3) / (n * DIM)
    loss = m + BETA * m

    zq = zq_flat.reshape(b, h, w_, c)
    zq = zp + lax.stop_gradient(zq - zp)
    zq = jnp.transpose(zq, (0, 3, 1, 2))
    return (zq, loss)


# speedup vs baseline: 1.1450x; 1.1450x over previous
import functools

import jax
import jax.numpy as jnp
from jax import lax
from jax.experimental import pallas as pl
from jax.experimental.pallas import tpu as pltpu
from jax.experimental.pallas import tpu_sc as plsc

N_CODES = 8192
DIM = 256
BETA = 0.25

R_BLK = 1024
C_BLK = 2048
C_HALF = 1024
N_CB = N_CODES // C_BLK


def _dist_argmin_body(zp_ref, w_ref, idx_ref, mind_ref,
                      zsq_ref, wsq_ref, best_ref, bidx_ref):
    rb = pl.program_id(0)
    cb = pl.program_id(1)
    zi = zp_ref[...]

    @pl.when(cb == 0)
    def _zsq():
        zsq_ref[...] = jnp.sum(zi * zi, axis=1, keepdims=True)

    @pl.when(rb == 0)
    def _wsq():
        wb_all = w_ref[...]
        wsq_ref[:, pl.ds(cb * C_BLK, C_BLK)] = jnp.sum(
            wb_all * wb_all, axis=1, keepdims=True).reshape(1, C_BLK)

    zsq = zsq_ref[...]
    colf = lax.broadcasted_iota(
        jnp.int32, (1, C_HALF), 1).astype(jnp.float32)

    for h in range(C_BLK // C_HALF):
        wb = w_ref[h * C_HALF:(h + 1) * C_HALF, :]
        s = lax.dot_general(zi, wb, (((1,), (1,)), ((), ())),
                            preferred_element_type=jnp.float32)
        wsq = wsq_ref[:, pl.ds(cb * C_BLK + h * C_HALF, C_HALF)]
        d = (zsq + wsq) - 2.0 * s

        bmin = jnp.min(d, axis=1, keepdims=True)
        bargf = (jnp.min(jnp.where(d == bmin, colf, float(N_CODES)),
                         axis=1, keepdims=True)
                 + (cb * C_BLK + h * C_HALF).astype(jnp.float32))

        if h == 0:
            @pl.when(cb == 0)
            def _init():
                best_ref[...] = bmin
                bidx_ref[...] = bargf

            @pl.when(cb > 0)
            def _update0():
                prev = best_ref[...]
                take = bmin < prev
                best_ref[...] = jnp.where(take, bmin, prev)
                bidx_ref[...] = jnp.where(take, bargf, bidx_ref[...])
        else:
            prev = best_ref[...]
            take = bmin < prev
            best_ref[...] = jnp.where(take, bmin, prev)
            bidx_ref[...] = jnp.where(take, bargf, bidx_ref[...])

    @pl.when(cb == N_CB - 1)
    def _emit():
        idx_ref[0] = bidx_ref[...].astype(jnp.int32)
        mind_ref[0] = best_ref[...]


def _dist_argmin(zp_flat, w):
    n_rb = zp_flat.shape[0] // R_BLK
    return pl.pallas_call(
        _dist_argmin_body,
        grid=(n_rb, N_CB),
        in_specs=[
            pl.BlockSpec((R_BLK, DIM), lambda rb, cb: (rb, 0)),
            pl.BlockSpec((C_BLK, DIM), lambda rb, cb: (cb, 0)),
        ],
        out_specs=[
            pl.BlockSpec((1, R_BLK, 1), lambda rb, cb: (rb, 0, 0)),
            pl.BlockSpec((1, R_BLK, 1), lambda rb, cb: (rb, 0, 0)),
        ],
        out_shape=[
            jax.ShapeDtypeStruct((n_rb, R_BLK, 1), jnp.int32),
            jax.ShapeDtypeStruct((n_rb, R_BLK, 1), jnp.float32),
        ],
        scratch_shapes=[
            pltpu.VMEM((R_BLK, 1), jnp.float32),
            pltpu.VMEM((1, N_CODES), jnp.float32),
            pltpu.VMEM((R_BLK, 1), jnp.float32),
            pltpu.VMEM((R_BLK, 1), jnp.float32),
        ],
        compiler_params=pltpu.CompilerParams(
            dimension_semantics=("parallel", "arbitrary")),
    )(zp_flat, w)


def _sc_gather(w, idx):
    info = plsc.get_sparse_core_info()
    nw = info.num_cores * info.num_subcores
    n = idx.shape[0]
    b_per_w = n // nw
    mesh = plsc.VectorSubcoreMesh(core_axis_name="c", subcore_axis_name="s")

    @functools.partial(
        pl.kernel, mesh=mesh,
        out_type=jax.ShapeDtypeStruct((n, DIM), jnp.float32),
        scratch_types=[
            pltpu.VMEM((b_per_w,), jnp.int32),
            pltpu.VMEM((b_per_w, DIM), jnp.float32),
            pltpu.SemaphoreType.DMA,
        ],
    )
    def gather_k(table_hbm, idx_hbm, out_hbm, idx_v, rows_v, sem):
        wid = lax.axis_index("s") * info.num_cores + lax.axis_index("c")
        base = wid * b_per_w
        pltpu.sync_copy(idx_hbm.at[pl.ds(base, b_per_w)], idx_v)
        pltpu.async_copy(table_hbm.at[idx_v], rows_v, sem).wait()
        pltpu.sync_copy(rows_v, out_hbm.at[pl.ds(base, b_per_w)])

    return gather_k(w, idx)


def kernel(z, W):
    b, c, h, w_ = z.shape
    n = b * h * w_
    zp = jnp.transpose(z, (0, 2, 3, 1))
    zp_flat = zp.reshape(n, DIM)

    idx3, mind3 = _dist_argmin(zp_flat, W)
    idx = idx3.reshape(n)

    zq_flat = _sc_gather(W, idx)

    m = jnp.sum(mind3) / (n * DIM)
    loss = m + BETA * m

    zq = zq_flat.reshape(b, h, w_, c)
    zq = zp + lax.stop_gradient(zq - zp)
    zq = jnp.transpose(zq, (0, 3, 1, 2))
    return (zq, loss)


# speedup vs baseline: 1.1513x; 1.0054x over previous
import functools

import jax
import jax.numpy as jnp
from jax import lax
from jax.experimental import pallas as pl
from jax.experimental.pallas import tpu as pltpu
from jax.experimental.pallas import tpu_sc as plsc

N_CODES = 8192
DIM = 256
BETA = 0.25

R_BLK = 1024
C_BLK = 2048
C_HALF = 1024
N_CB = N_CODES // C_BLK


def _dist_argmin_body(z_ref, w_ref, idx_ref, mind_ref,
                      zi_ref, zsq_ref, wsq_ref, best_ref, bidx_ref):
    rb = pl.program_id(0)
    cb = pl.program_id(1)

    @pl.when(cb == 0)
    def _zsq():
        zi_t = jnp.transpose(z_ref[0], (1, 0))
        zi_ref[...] = zi_t
        zsq_ref[...] = jnp.sum(zi_t * zi_t, axis=1, keepdims=True)

    zi = zi_ref[...]

    @pl.when(rb == 0)
    def _wsq():
        wb_all = w_ref[...]
        wsq_ref[:, pl.ds(cb * C_BLK, C_BLK)] = jnp.sum(
            wb_all * wb_all, axis=1, keepdims=True).reshape(1, C_BLK)

    zsq = zsq_ref[...]
    colf = lax.broadcasted_iota(
        jnp.int32, (1, C_HALF), 1).astype(jnp.float32)

    for h in range(C_BLK // C_HALF):
        wb = w_ref[h * C_HALF:(h + 1) * C_HALF, :]
        s = lax.dot_general(zi, wb, (((1,), (1,)), ((), ())),
                            preferred_element_type=jnp.float32)
        wsq = wsq_ref[:, pl.ds(cb * C_BLK + h * C_HALF, C_HALF)]
        d = (zsq + wsq) - 2.0 * s

        bmin = jnp.min(d, axis=1, keepdims=True)
        bargf = (jnp.min(jnp.where(d == bmin, colf, float(N_CODES)),
                         axis=1, keepdims=True)
                 + (cb * C_BLK + h * C_HALF).astype(jnp.float32))

        if h == 0:
            @pl.when(cb == 0)
            def _init():
                best_ref[...] = bmin
                bidx_ref[...] = bargf

            @pl.when(cb > 0)
            def _update0():
                prev = best_ref[...]
                take = bmin < prev
                best_ref[...] = jnp.where(take, bmin, prev)
                bidx_ref[...] = jnp.where(take, bargf, bidx_ref[...])
        else:
            prev = best_ref[...]
            take = bmin < prev
            best_ref[...] = jnp.where(take, bmin, prev)
            bidx_ref[...] = jnp.where(take, bargf, bidx_ref[...])

    @pl.when(cb == N_CB - 1)
    def _emit():
        idx_ref[0] = bidx_ref[...].astype(jnp.int32)
        mind_ref[0] = best_ref[...]


def _dist_argmin(z3, w):
    n_rb = z3.shape[0] * z3.shape[2] // R_BLK
    return pl.pallas_call(
        _dist_argmin_body,
        grid=(n_rb, N_CB),
        in_specs=[
            pl.BlockSpec((1, DIM, R_BLK), lambda rb, cb: (rb, 0, 0)),
            pl.BlockSpec((C_BLK, DIM), lambda rb, cb: (cb, 0)),
        ],
        out_specs=[
            pl.BlockSpec((1, R_BLK, 1), lambda rb, cb: (rb, 0, 0)),
            pl.BlockSpec((1, R_BLK, 1), lambda rb, cb: (rb, 0, 0)),
        ],
        out_shape=[
            jax.ShapeDtypeStruct((n_rb, R_BLK, 1), jnp.int32),
            jax.ShapeDtypeStruct((n_rb, R_BLK, 1), jnp.float32),
        ],
        scratch_shapes=[
            pltpu.VMEM((R_BLK, DIM), jnp.float32),
            pltpu.VMEM((R_BLK, 1), jnp.float32),
            pltpu.VMEM((1, N_CODES), jnp.float32),
            pltpu.VMEM((R_BLK, 1), jnp.float32),
            pltpu.VMEM((R_BLK, 1), jnp.float32),
        ],
        compiler_params=pltpu.CompilerParams(
            dimension_semantics=("parallel", "arbitrary")),
    )(z3, w)


def _sc_gather(w, idx):
    info = plsc.get_sparse_core_info()
    nw = info.num_cores * info.num_subcores
    n = idx.shape[0]
    b_per_w = n // nw
    mesh = plsc.VectorSubcoreMesh(core_axis_name="c", subcore_axis_name="s")

    @functools.partial(
        pl.kernel, mesh=mesh,
        out_type=jax.ShapeDtypeStruct((n, DIM), jnp.float32),
        scratch_types=[
            pltpu.VMEM((b_per_w,), jnp.int32),
            pltpu.VMEM((b_per_w, DIM), jnp.float32),
            pltpu.SemaphoreType.DMA,
        ],
    )
    def gather_k(table_hbm, idx_hbm, out_hbm, idx_v, rows_v, sem):
        wid = lax.axis_index("s") * info.num_cores + lax.axis_index("c")
        base = wid * b_per_w
        pltpu.sync_copy(idx_hbm.at[pl.ds(base, b_per_w)], idx_v)
        pltpu.async_copy(table_hbm.at[idx_v], rows_v, sem).wait()
        pltpu.sync_copy(rows_v, out_hbm.at[pl.ds(base, b_per_w)])

    return gather_k(w, idx)


def kernel(z, W):
    b, c, h, w_ = z.shape
    n = b * h * w_
    z3 = z.reshape(b, c, h * w_)

    idx3, mind3 = _dist_argmin(z3, W)
    idx = idx3.reshape(n)

    zq_flat = _sc_gather(W, idx)

    m = jnp.sum(mind3) / (n * DIM)
    loss = m + BETA * m

    zq = jnp.transpose(zq_flat.reshape(b, h, w_, c), (0, 3, 1, 2))
    return (zq, loss)


# speedup vs baseline: 1.2249x; 1.0639x over previous
import functools

import jax
import jax.numpy as jnp
from jax import lax
from jax.experimental import pallas as pl
from jax.experimental.pallas import tpu as pltpu
from jax.experimental.pallas import tpu_sc as plsc

N_CODES = 8192
DIM = 256
BETA = 0.25

R_BLK = 1024
C_BLK = 4096
C_HALF = 1024
N_CB = N_CODES // C_BLK


def _dist_argmin_body(z_ref, w_ref, idx_ref, lsum_ref,
                      zi_ref, zsq_ref, wsq_ref, best_ref, bidx_ref):
    rb = pl.program_id(0)
    cb = pl.program_id(1)

    @pl.when(cb == 0)
    def _zsq():
        zi_t = jnp.transpose(z_ref[0], (1, 0))
        zi_ref[...] = zi_t
        zsq_ref[...] = jnp.sum(zi_t * zi_t, axis=1, keepdims=True)

    zi = zi_ref[...]

    @pl.when(rb == 0)
    def _wsq():
        wb_all = w_ref[...]
        wsq_ref[:, pl.ds(cb * C_BLK, C_BLK)] = jnp.sum(
            wb_all * wb_all, axis=1, keepdims=True).reshape(1, C_BLK)

    zsq = zsq_ref[...]
    colf = lax.broadcasted_iota(
        jnp.int32, (1, C_HALF), 1).astype(jnp.float32)

    for h in range(C_BLK // C_HALF):
        wb = w_ref[h * C_HALF:(h + 1) * C_HALF, :]
        s = lax.dot_general(zi, wb, (((1,), (1,)), ((), ())),
                            preferred_element_type=jnp.float32)
        wsq = wsq_ref[:, pl.ds(cb * C_BLK + h * C_HALF, C_HALF)]
        d = (zsq + wsq) - 2.0 * s

        bmin = jnp.min(d, axis=1, keepdims=True)
        bargf = (jnp.min(jnp.where(d == bmin, colf, float(N_CODES)),
                         axis=1, keepdims=True)
                 + (cb * C_BLK + h * C_HALF).astype(jnp.float32))

        if h == 0:
            @pl.when(cb == 0)
            def _init():
                best_ref[...] = bmin
                bidx_ref[...] = bargf

            @pl.when(cb > 0)
            def _update0():
                prev = best_ref[...]
                take = bmin < prev
                best_ref[...] = jnp.where(take, bmin, prev)
                bidx_ref[...] = jnp.where(take, bargf, bidx_ref[...])
        else:
            prev = best_ref[...]
            take = bmin < prev
            best_ref[...] = jnp.where(take, bmin, prev)
            bidx_ref[...] = jnp.where(take, bargf, bidx_ref[...])

    @pl.when(cb == N_CB - 1)
    def _emit():
        idx_ref[0] = bidx_ref[...].astype(jnp.int32)
        row_sum = jnp.sum(best_ref[...])
        prev_sum = jnp.where(rb == 0, 0.0, lsum_ref[0, 0])
        lsum_ref[0, 0] = prev_sum + row_sum


def _dist_argmin(z3, w):
    n_rb = z3.shape[0] * z3.shape[2] // R_BLK
    return pl.pallas_call(
        _dist_argmin_body,
        grid=(n_rb, N_CB),
        in_specs=[
            pl.BlockSpec((1, DIM, R_BLK), lambda rb, cb: (rb, 0, 0)),
            pl.BlockSpec((C_BLK, DIM), lambda rb, cb: (cb, 0)),
        ],
        out_specs=[
            pl.BlockSpec((1, R_BLK, 1), lambda rb, cb: (rb, 0, 0)),
            pl.BlockSpec(memory_space=pltpu.MemorySpace.SMEM),
        ],
        out_shape=[
            jax.ShapeDtypeStruct((n_rb, R_BLK, 1), jnp.int32),
            jax.ShapeDtypeStruct((1, 1), jnp.float32),
        ],
        scratch_shapes=[
            pltpu.VMEM((R_BLK, DIM), jnp.float32),
            pltpu.VMEM((R_BLK, 1), jnp.float32),
            pltpu.VMEM((1, N_CODES), jnp.float32),
            pltpu.VMEM((R_BLK, 1), jnp.float32),
            pltpu.VMEM((R_BLK, 1), jnp.float32),
        ],
        compiler_params=pltpu.CompilerParams(
            dimension_semantics=("arbitrary", "arbitrary")),
    )(z3, w)


def _sc_gather(w, idx):
    info = plsc.get_sparse_core_info()
    nw = info.num_cores * info.num_subcores
    n = idx.shape[0]
    b_per_w = n // nw
    mesh = plsc.VectorSubcoreMesh(core_axis_name="c", subcore_axis_name="s")

    @functools.partial(
        pl.kernel, mesh=mesh,
        out_type=jax.ShapeDtypeStruct((n, DIM), jnp.float32),
        scratch_types=[
            pltpu.VMEM((b_per_w,), jnp.int32),
            pltpu.VMEM((b_per_w, DIM), jnp.float32),
            pltpu.SemaphoreType.DMA,
        ],
    )
    def gather_k(table_hbm, idx_hbm, out_hbm, idx_v, rows_v, sem):
        wid = lax.axis_index("s") * info.num_cores + lax.axis_index("c")
        base = wid * b_per_w
        pltpu.sync_copy(idx_hbm.at[pl.ds(base, b_per_w)], idx_v)
        pltpu.async_copy(table_hbm.at[idx_v], rows_v, sem).wait()
        pltpu.sync_copy(rows_v, out_hbm.at[pl.ds(base, b_per_w)])

    return gather_k(w, idx)


def kernel(z, W):
    b, c, h, w_ = z.shape
    n = b * h * w_
    z3 = z.reshape(b, c, h * w_)

    idx3, lsum = _dist_argmin(z3, W)
    idx = idx3.reshape(n)

    zq_flat = _sc_gather(W, idx)

    m = lsum[0, 0] / (n * DIM)
    loss = m + BETA * m

    zq = jnp.transpose(zq_flat.reshape(b, h, w_, c), (0, 3, 1, 2))
    return (zq, loss)


# speedup vs baseline: 1.2761x; 1.0418x over previous
import functools

import jax
import jax.numpy as jnp
from jax import lax
from jax.experimental import pallas as pl
from jax.experimental.pallas import tpu as pltpu
from jax.experimental.pallas import tpu_sc as plsc

N_CODES = 8192
DIM = 256
BETA = 0.25

R_BLK = 2048
C_BLK = 4096
C_HALF = 1024
N_CB = N_CODES // C_BLK


def _dist_argmin_body(z_ref, w_ref, idx_ref, lsum_ref,
                      zi_ref, zsq_ref, wsq_ref, best_ref, bidx_ref):
    rb = pl.program_id(0)
    cb = pl.program_id(1)

    @pl.when(cb == 0)
    def _zsq():
        zi_t = jnp.transpose(z_ref[...], (0, 2, 1)).reshape(R_BLK, DIM)
        zi_ref[...] = zi_t
        zsq_ref[...] = jnp.sum(zi_t * zi_t, axis=1, keepdims=True)

    zi = zi_ref[...]

    @pl.when(rb == 0)
    def _wsq():
        wb_all = w_ref[...]
        wsq_ref[:, pl.ds(cb * C_BLK, C_BLK)] = jnp.sum(
            wb_all * wb_all, axis=1, keepdims=True).reshape(1, C_BLK)

    zsq = zsq_ref[...]
    colf = lax.broadcasted_iota(
        jnp.int32, (1, C_HALF), 1).astype(jnp.float32)

    for h in range(C_BLK // C_HALF):
        wb = w_ref[h * C_HALF:(h + 1) * C_HALF, :]
        s = lax.dot_general(zi, wb, (((1,), (1,)), ((), ())),
                            preferred_element_type=jnp.float32)
        wsq = wsq_ref[:, pl.ds(cb * C_BLK + h * C_HALF, C_HALF)]
        d = (zsq + wsq) - 2.0 * s

        bmin = jnp.min(d, axis=1, keepdims=True)
        bargf = (jnp.min(jnp.where(d == bmin, colf, float(N_CODES)),
                         axis=1, keepdims=True)
                 + (cb * C_BLK + h * C_HALF).astype(jnp.float32))

        if h == 0:
            @pl.when(cb == 0)
            def _init():
                best_ref[...] = bmin
                bidx_ref[...] = bargf

            @pl.when(cb > 0)
            def _update0():
                prev = best_ref[...]
                take = bmin < prev
                best_ref[...] = jnp.where(take, bmin, prev)
                bidx_ref[...] = jnp.where(take, bargf, bidx_ref[...])
        else:
            prev = best_ref[...]
            take = bmin < prev
            best_ref[...] = jnp.where(take, bmin, prev)
            bidx_ref[...] = jnp.where(take, bargf, bidx_ref[...])

    @pl.when(cb == N_CB - 1)
    def _emit():
        idx_ref[0] = bidx_ref[...].astype(jnp.int32)
        row_sum = jnp.sum(best_ref[...])
        prev_sum = jnp.where(rb == 0, 0.0, lsum_ref[0, 0])
        lsum_ref[0, 0] = prev_sum + row_sum


def _dist_argmin(z3, w):
    n_rb = z3.shape[0] * z3.shape[2] // R_BLK
    return pl.pallas_call(
        _dist_argmin_body,
        grid=(n_rb, N_CB),
        in_specs=[
            pl.BlockSpec((R_BLK // 1024, DIM, 1024), lambda rb, cb: (rb, 0, 0)),
            pl.BlockSpec((C_BLK, DIM), lambda rb, cb: (cb, 0)),
        ],
        out_specs=[
            pl.BlockSpec((1, R_BLK, 1), lambda rb, cb: (rb, 0, 0)),
            pl.BlockSpec(memory_space=pltpu.MemorySpace.SMEM),
        ],
        out_shape=[
            jax.ShapeDtypeStruct((n_rb, R_BLK, 1), jnp.int32),
            jax.ShapeDtypeStruct((1, 1), jnp.float32),
        ],
        scratch_shapes=[
            pltpu.VMEM((R_BLK, DIM), jnp.float32),
            pltpu.VMEM((R_BLK, 1), jnp.float32),
            pltpu.VMEM((1, N_CODES), jnp.float32),
            pltpu.VMEM((R_BLK, 1), jnp.float32),
            pltpu.VMEM((R_BLK, 1), jnp.float32),
        ],
        compiler_params=pltpu.CompilerParams(
            dimension_semantics=("arbitrary", "arbitrary")),
    )(z3, w)


def _sc_gather(w, idx):
    info = plsc.get_sparse_core_info()
    nw = info.num_cores * info.num_subcores
    n = idx.shape[0]
    b_per_w = n // nw
    mesh = plsc.VectorSubcoreMesh(core_axis_name="c", subcore_axis_name="s")

    @functools.partial(
        pl.kernel, mesh=mesh,
        out_type=jax.ShapeDtypeStruct((n, DIM), jnp.float32),
        scratch_types=[
            pltpu.VMEM((b_per_w,), jnp.int32),
            pltpu.VMEM((b_per_w, DIM), jnp.float32),
            pltpu.SemaphoreType.DMA,
        ],
    )
    def gather_k(table_hbm, idx_hbm, out_hbm, idx_v, rows_v, sem):
        wid = lax.axis_index("s") * info.num_cores + lax.axis_index("c")
        base = wid * b_per_w
        pltpu.sync_copy(idx_hbm.at[pl.ds(base, b_per_w)], idx_v)
        pltpu.async_copy(table_hbm.at[idx_v], rows_v, sem).wait()
        pltpu.sync_copy(rows_v, out_hbm.at[pl.ds(base, b_per_w)])

    return gather_k(w, idx)


def kernel(z, W):
    b, c, h, w_ = z.shape
    n = b * h * w_
    z3 = z.reshape(b, c, h * w_)

    idx3, lsum = _dist_argmin(z3, W)
    idx = idx3.reshape(n)

    zq_flat = _sc_gather(W, idx)

    m = lsum[0, 0] / (n * DIM)
    loss = m + BETA * m

    zq = jnp.transpose(zq_flat.reshape(b, h, w_, c), (0, 3, 1, 2))
    return (zq, loss)


# speedup vs baseline: 1.3209x; 1.0351x over previous
import functools

import jax
import jax.numpy as jnp
from jax import lax
from jax.experimental import pallas as pl
from jax.experimental.pallas import tpu as pltpu
from jax.experimental.pallas import tpu_sc as plsc

N_CODES = 8192
DIM = 256
BETA = 0.25

R_BLK = 4096
C_BLK = 4096
C_HALF = 1024
N_CB = N_CODES // C_BLK


def _dist_argmin_body(z_ref, w_ref, idx_ref, lsum_ref,
                      zi_ref, zsq_ref, wsq_ref, best_ref, bidx_ref):
    rb = pl.program_id(0)
    cb = pl.program_id(1)

    @pl.when(cb == 0)
    def _zsq():
        zi_t = jnp.transpose(z_ref[...], (0, 2, 1)).reshape(R_BLK, DIM)
        zi_ref[...] = zi_t
        zsq_ref[...] = jnp.sum(zi_t * zi_t, axis=1, keepdims=True)

    zi = zi_ref[...]

    @pl.when(rb == 0)
    def _wsq():
        wb_all = w_ref[...]
        wsq_ref[:, pl.ds(cb * C_BLK, C_BLK)] = jnp.sum(
            wb_all * wb_all, axis=1, keepdims=True).reshape(1, C_BLK)

    zsq = zsq_ref[...]
    colf = lax.broadcasted_iota(
        jnp.int32, (1, C_HALF), 1).astype(jnp.float32)

    for h in range(C_BLK // C_HALF):
        wb = w_ref[h * C_HALF:(h + 1) * C_HALF, :]
        s = lax.dot_general(zi, wb, (((1,), (1,)), ((), ())),
                            preferred_element_type=jnp.float32)
        wsq = wsq_ref[:, pl.ds(cb * C_BLK + h * C_HALF, C_HALF)]
        d = (zsq + wsq) - 2.0 * s

        bmin = jnp.min(d, axis=1, keepdims=True)
        bargf = (jnp.min(jnp.where(d == bmin, colf, float(N_CODES)),
                         axis=1, keepdims=True)
                 + (cb * C_BLK + h * C_HALF).astype(jnp.float32))

        if h == 0:
            @pl.when(cb == 0)
            def _init():
                best_ref[...] = bmin
                bidx_ref[...] = bargf

            @pl.when(cb > 0)
            def _update0():
                prev = best_ref[...]
                take = bmin < prev
                best_ref[...] = jnp.where(take, bmin, prev)
                bidx_ref[...] = jnp.where(take, bargf, bidx_ref[...])
        else:
            prev = best_ref[...]
            take = bmin < prev
            best_ref[...] = jnp.where(take, bmin, prev)
            bidx_ref[...] = jnp.where(take, bargf, bidx_ref[...])

    @pl.when(cb == N_CB - 1)
    def _emit():
        idx_ref[0] = bidx_ref[...].astype(jnp.int32)
        row_sum = jnp.sum(best_ref[...])
        prev_sum = jnp.where(rb == 0, 0.0, lsum_ref[0, 0])
        lsum_ref[0, 0] = prev_sum + row_sum


def _dist_argmin(z3, w):
    n_rb = z3.shape[0] * z3.shape[2] // R_BLK
    return pl.pallas_call(
        _dist_argmin_body,
        grid=(n_rb, N_CB),
        in_specs=[
            pl.BlockSpec((R_BLK // 1024, DIM, 1024), lambda rb, cb: (rb, 0, 0)),
            pl.BlockSpec((C_BLK, DIM), lambda rb, cb: (cb, 0)),
        ],
        out_specs=[
            pl.BlockSpec((1, R_BLK, 1), lambda rb, cb: (rb, 0, 0)),
            pl.BlockSpec(memory_space=pltpu.MemorySpace.SMEM),
        ],
        out_shape=[
            jax.ShapeDtypeStruct((n_rb, R_BLK, 1), jnp.int32),
            jax.ShapeDtypeStruct((1, 1), jnp.float32),
        ],
        scratch_shapes=[
            pltpu.VMEM((R_BLK, DIM), jnp.float32),
            pltpu.VMEM((R_BLK, 1), jnp.float32),
            pltpu.VMEM((1, N_CODES), jnp.float32),
            pltpu.VMEM((R_BLK, 1), jnp.float32),
            pltpu.VMEM((R_BLK, 1), jnp.float32),
        ],
        compiler_params=pltpu.CompilerParams(
            dimension_semantics=("arbitrary", "arbitrary")),
    )(z3, w)


def _sc_gather(w, idx):
    info = plsc.get_sparse_core_info()
    nw = info.num_cores * info.num_subcores
    n = idx.shape[0]
    b_per_w = n // nw
    mesh = plsc.VectorSubcoreMesh(core_axis_name="c", subcore_axis_name="s")

    @functools.partial(
        pl.kernel, mesh=mesh,
        out_type=jax.ShapeDtypeStruct((n, DIM), jnp.float32),
        scratch_types=[
            pltpu.VMEM((b_per_w,), jnp.int32),
            pltpu.VMEM((b_per_w, DIM), jnp.float32),
            pltpu.SemaphoreType.DMA,
        ],
    )
    def gather_k(table_hbm, idx_hbm, out_hbm, idx_v, rows_v, sem):
        wid = lax.axis_index("s") * info.num_cores + lax.axis_index("c")
        base = wid * b_per_w
        pltpu.sync_copy(idx_hbm.at[pl.ds(base, b_per_w)], idx_v)
        pltpu.async_copy(table_hbm.at[idx_v], rows_v, sem).wait()
        pltpu.sync_copy(rows_v, out_hbm.at[pl.ds(base, b_per_w)])

    return gather_k(w, idx)


def kernel(z, W):
    b, c, h, w_ = z.shape
    n = b * h * w_
    z3 = z.reshape(b, c, h * w_)

    idx3, lsum = _dist_argmin(z3, W)
    idx = idx3.reshape(n)

    zq_flat = _sc_gather(W, idx)

    m = lsum[0, 0] / (n * DIM)
    loss = m + BETA * m

    zq = jnp.transpose(zq_flat.reshape(b, h, w_, c), (0, 3, 1, 2))
    return (zq, loss)


# speedup vs baseline: 1.3283x; 1.0056x over previous
import functools

import jax
import jax.numpy as jnp
from jax import lax
from jax.experimental import pallas as pl
from jax.experimental.pallas import tpu as pltpu
from jax.experimental.pallas import tpu_sc as plsc

N_CODES = 8192
DIM = 256
BETA = 0.25

R_BLK = 4096
C_BLK = 4096
C_HALF = 1024
N_CB = N_CODES // C_BLK


def _dist_argmin_body(z_ref, w_ref, idx_ref, lsum_ref,
                      zi_ref, zsq_ref, wsq_ref, best_ref, bidx_ref):
    rb = pl.program_id(0)
    cb = pl.program_id(1)

    @pl.when(cb == 0)
    def _zsq():
        zi_t = jnp.transpose(z_ref[...], (0, 2, 1)).reshape(R_BLK, DIM)
        zi_ref[...] = zi_t
        zsq_ref[...] = jnp.sum(zi_t * zi_t, axis=1, keepdims=True)

    zi = zi_ref[...]

    @pl.when(rb == 0)
    def _wsq():
        wb_all = w_ref[...]
        wsq_ref[:, pl.ds(cb * C_BLK, C_BLK)] = jnp.sum(
            wb_all * wb_all, axis=1, keepdims=True).reshape(1, C_BLK)

    zsq = zsq_ref[...]
    colf = lax.broadcasted_iota(
        jnp.int32, (1, C_HALF), 1).astype(jnp.float32)

    for h in range(C_BLK // C_HALF):
        wb = w_ref[h * C_HALF:(h + 1) * C_HALF, :]
        s = lax.dot_general(zi, wb, (((1,), (1,)), ((), ())),
                            preferred_element_type=jnp.float32)
        wsq = wsq_ref[:, pl.ds(cb * C_BLK + h * C_HALF, C_HALF)]
        d = (zsq + wsq) - 2.0 * s

        bmin = jnp.min(d, axis=1, keepdims=True)
        bargf = (jnp.min(jnp.where(d == bmin, colf, float(N_CODES)),
                         axis=1, keepdims=True)
                 + (cb * C_BLK + h * C_HALF).astype(jnp.float32))

        if h == 0:
            @pl.when(cb == 0)
            def _init():
                best_ref[...] = bmin
                bidx_ref[...] = bargf

            @pl.when(cb > 0)
            def _update0():
                prev = best_ref[...]
                take = bmin < prev
                best_ref[...] = jnp.where(take, bmin, prev)
                bidx_ref[...] = jnp.where(take, bargf, bidx_ref[...])
        else:
            prev = best_ref[...]
            take = bmin < prev
            best_ref[...] = jnp.where(take, bmin, prev)
            bidx_ref[...] = jnp.where(take, bargf, bidx_ref[...])

    @pl.when(cb == N_CB - 1)
    def _emit():
        idx_ref[0] = bidx_ref[...].astype(jnp.int32)
        row_sum = jnp.sum(best_ref[...])
        total = jnp.where(rb == 0, 0.0, lsum_ref[0, 0]) + row_sum
        m = total * (1.0 / (N_CODES * DIM))
        lsum_ref[0, 0] = jnp.where(rb == pl.num_programs(0) - 1,
                                   m + BETA * m, total)


def _dist_argmin(z3, w):
    n_rb = z3.shape[0] * z3.shape[2] // R_BLK
    return pl.pallas_call(
        _dist_argmin_body,
        grid=(n_rb, N_CB),
        in_specs=[
            pl.BlockSpec((R_BLK // 1024, DIM, 1024), lambda rb, cb: (rb, 0, 0)),
            pl.BlockSpec((C_BLK, DIM), lambda rb, cb: (cb, 0)),
        ],
        out_specs=[
            pl.BlockSpec((1, R_BLK, 1), lambda rb, cb: (rb, 0, 0)),
            pl.BlockSpec(memory_space=pltpu.MemorySpace.SMEM),
        ],
        out_shape=[
            jax.ShapeDtypeStruct((n_rb, R_BLK, 1), jnp.int32),
            jax.ShapeDtypeStruct((1, 1), jnp.float32),
        ],
        scratch_shapes=[
            pltpu.VMEM((R_BLK, DIM), jnp.float32),
            pltpu.VMEM((R_BLK, 1), jnp.float32),
            pltpu.VMEM((1, N_CODES), jnp.float32),
            pltpu.VMEM((R_BLK, 1), jnp.float32),
            pltpu.VMEM((R_BLK, 1), jnp.float32),
        ],
        compiler_params=pltpu.CompilerParams(
            dimension_semantics=("arbitrary", "arbitrary")),
    )(z3, w)


def _sc_gather(w, idx):
    info = plsc.get_sparse_core_info()
    nw = info.num_cores * info.num_subcores
    n = idx.shape[0]
    b_per_w = n // nw
    mesh = plsc.VectorSubcoreMesh(core_axis_name="c", subcore_axis_name="s")

    @functools.partial(
        pl.kernel, mesh=mesh,
        out_type=jax.ShapeDtypeStruct((n, DIM), jnp.float32),
        scratch_types=[
            pltpu.VMEM((b_per_w,), jnp.int32),
            pltpu.VMEM((b_per_w, DIM), jnp.float32),
            pltpu.SemaphoreType.DMA,
        ],
    )
    def gather_k(table_hbm, idx_hbm, out_hbm, idx_v, rows_v, sem):
        wid = lax.axis_index("s") * info.num_cores + lax.axis_index("c")
        base = wid * b_per_w
        pltpu.sync_copy(idx_hbm.at[pl.ds(base, b_per_w)], idx_v)
        pltpu.async_copy(table_hbm.at[idx_v], rows_v, sem).wait()
        pltpu.sync_copy(rows_v, out_hbm.at[pl.ds(base, b_per_w)])

    return gather_k(w, idx)


def kernel(z, W):
    b, c, h, w_ = z.shape
    n = b * h * w_
    z3 = z.reshape(b, c, h * w_)

    idx3, lsum = _dist_argmin(z3, W)
    idx = idx3.reshape(n)

    zq_flat = _sc_gather(W, idx)

    loss = lsum[0, 0]

    zq = jnp.transpose(zq_flat.reshape(b, h, w_, c), (0, 3, 1, 2))
    return (zq, loss)
